# initial kernel scaffold (unmeasured)
import jax
import jax.numpy as jnp
from jax import lax
from jax.experimental import pallas as pl
from jax.experimental.pallas import tpu as pltpu


def kernel(
    x,
):
    def body(*refs):
        pass

    out_shape = jax.ShapeDtypeStruct(..., jnp.float32)
    return pl.pallas_call(body, out_shape=out_shape)(...)


# baseline (device time: 324778 ns/iter reference)
import jax
import jax.numpy as jnp
from jax import lax
from jax.experimental import pallas as pl
from jax.experimental.pallas import tpu as pltpu

NZ = 4
M = 4096
N = 4096
CHUNK = N // NZ


def kernel(x):
    x = x.reshape(M, N).astype(jnp.bfloat16)

    def body(x_ref, out_ref, comm_ref, p1_ref, p2_ref,
             send_sems, recv_sems, local_sems):
        xi = lax.axis_index("x")
        yi = lax.axis_index("y")
        zi = lax.axis_index("z")
        right = (zi + 1) % NZ
        left = (zi - 1) % NZ

        barrier_sem = pltpu.get_barrier_semaphore()
        for nbr in (left, right):
            pl.semaphore_signal(
                barrier_sem, inc=1,
                device_id=(xi, yi, nbr),
                device_id_type=pl.DeviceIdType.MESH,
            )
        pl.semaphore_wait(barrier_sem, 2)

        c0 = (zi + 3) % NZ
        c1 = (zi + 2) % NZ
        c2 = (zi + 1) % NZ

        ld1 = pltpu.make_async_copy(
            x_ref.at[:, pl.ds(c1 * CHUNK, CHUNK)], p1_ref, local_sems.at[0])
        ld2 = pltpu.make_async_copy(
            x_ref.at[:, pl.ds(c2 * CHUNK, CHUNK)], p2_ref, local_sems.at[1])
        ld3 = pltpu.make_async_copy(
            x_ref.at[:, pl.ds(zi * CHUNK, CHUNK)], out_ref, local_sems.at[2])
        ld1.start()
        ld2.start()
        ld3.start()

        rdma0 = pltpu.make_async_remote_copy(
            src_ref=x_ref.at[:, pl.ds(c0 * CHUNK, CHUNK)],
            dst_ref=comm_ref.at[0],
            send_sem=send_sems.at[0],
            recv_sem=recv_sems.at[0],
            device_id=(xi, yi, right),
            device_id_type=pl.DeviceIdType.MESH,
        )
        rdma0.start()
        rdma0.wait()

        ld1.wait()
        p1_ref[...] = p1_ref[...] + comm_ref[0]
        rdma1 = pltpu.make_async_remote_copy(
            src_ref=p1_ref,
            dst_ref=comm_ref.at[1],
            send_sem=send_sems.at[1],
            recv_sem=recv_sems.at[1],
            device_id=(xi, yi, right),
            device_id_type=pl.DeviceIdType.MESH,
        )
        rdma1.start()
        rdma1.wait()

        ld2.wait()
        p2_ref[...] = p2_ref[...] + comm_ref[1]
        rdma2 = pltpu.make_async_remote_copy(
            src_ref=p2_ref,
            dst_ref=comm_ref.at[2],
            send_sem=send_sems.at[2],
            recv_sem=recv_sems.at[2],
            device_id=(xi, yi, right),
            device_id_type=pl.DeviceIdType.MESH,
        )
        rdma2.start()
        rdma2.wait()

        ld3.wait()
        out_ref[...] = out_ref[...] + comm_ref[2]

    return pl.pallas_call(
        body,
        out_shape=jax.ShapeDtypeStruct((M, CHUNK), jnp.bfloat16),
        in_specs=[pl.BlockSpec(memory_space=pltpu.MemorySpace.HBM)],
        out_specs=pl.BlockSpec(memory_space=pltpu.VMEM),
        scratch_shapes=[
            pltpu.VMEM((3, M, CHUNK), jnp.bfloat16),
            pltpu.VMEM((M, CHUNK), jnp.bfloat16),
            pltpu.VMEM((M, CHUNK), jnp.bfloat16),
            pltpu.SemaphoreType.DMA((3,)),
            pltpu.SemaphoreType.DMA((3,)),
            pltpu.SemaphoreType.DMA((3,)),
        ],
        compiler_params=pltpu.CompilerParams(
            collective_id=0,
            vmem_limit_bytes=100 * 1024 * 1024,
        ),
    )(x)
